# Initial kernel scaffold; baseline (speedup 1.0000x reference)
import jax
import jax.numpy as jnp
from jax.experimental import pallas as pl


def kernel(x, memory):
    raise NotImplementedError("write your pallas kernel here")


# speedup vs baseline: 1.0043x; 1.0043x over previous
import jax
import jax.numpy as jnp
from jax.experimental import pallas as pl
from jax.experimental.pallas import tpu as pltpu

_T = 0.05
_BLK = 1024


def _mm_kernel(x_ref, m_ref, o_ref):
    acc = jax.lax.dot_general(
        x_ref[...], m_ref[...],
        dimension_numbers=(((1,), (1,)), ((), ())),
        preferred_element_type=jnp.float32)
    o_ref[...] = acc / _T


def kernel(x, memory):
    b, k = x.shape
    n = memory.shape[0]
    return pl.pallas_call(
        _mm_kernel,
        grid=(pl.cdiv(n, _BLK),),
        in_specs=[
            pl.BlockSpec((b, k), lambda i: (0, 0)),
            pl.BlockSpec((_BLK, k), lambda i: (i, 0)),
        ],
        out_specs=pl.BlockSpec((b, _BLK), lambda i: (0, i)),
        out_shape=jax.ShapeDtypeStruct((b, n), jnp.float32),
        compiler_params=pltpu.CompilerParams(
            dimension_semantics=("arbitrary",)),
    )(x, memory)


# speedup vs baseline: 1.0056x; 1.0013x over previous
import jax
import jax.numpy as jnp
from jax.experimental import pallas as pl
from jax.experimental.pallas import tpu as pltpu

_T = 0.05
_BLK = 2048


def _mm_kernel(x_ref, m_ref, o_ref):
    acc = jax.lax.dot_general(
        x_ref[...], m_ref[...],
        dimension_numbers=(((1,), (1,)), ((), ())),
        preferred_element_type=jnp.float32)
    o_ref[...] = acc / _T


def kernel(x, memory):
    b, k = x.shape
    n = memory.shape[0]
    return pl.pallas_call(
        _mm_kernel,
        grid=(pl.cdiv(n, _BLK),),
        in_specs=[
            pl.BlockSpec((b, k), lambda i: (0, 0)),
            pl.BlockSpec((_BLK, k), lambda i: (i, 0)),
        ],
        out_specs=pl.BlockSpec((b, _BLK), lambda i: (0, i)),
        out_shape=jax.ShapeDtypeStruct((b, n), jnp.float32),
        compiler_params=pltpu.CompilerParams(
            dimension_semantics=("arbitrary",)),
    )(x, memory)
